# baseline (speedup 1.0000x reference)
import jax
import jax.numpy as jnp
from jax import lax
from jax.experimental import pallas as pl
from jax.experimental.pallas import tpu as pltpu
from jax.experimental.pallas import tpu_sc as plsc

B, C, H, W = 4, 3, 384, 384
HW = H * W
NCORES, NSUB, L = 2, 16, 16
ROWS_PER_TILE = H // NSUB
TILE_ELEMS = ROWS_PER_TILE * W
HALO = 8
WIN = ROWS_PER_TILE + 2 * HALO
FB_CHUNK = 32
GROUPS = W // L
UNROLL = 4
MAGIC = 12582912.0


def _rne(x):
    return (x + MAGIC) - MAGIC


def _take16(x, idx):
    return lax.gather(
        x, idx[:, None],
        lax.GatherDimensionNumbers(
            offset_dims=(), collapsed_slice_dims=(0,), start_index_map=(0,)),
        slice_sizes=(1,),
        mode=lax.GatherScatterMode.PROMISE_IN_BOUNDS)


def _sc_warp_body(x_hbm, out_hbm, cp_hbm, fxw, fyw, imw, vr, vg, vb,
                  flg_v, flg_all, shared, sem_win, sem_w, sem_cp):
    core = lax.axis_index("c")
    sub = lax.axis_index("s")
    row0 = sub * ROWS_PER_TILE
    row0_f = row0.astype(jnp.float32)
    lane = lax.iota(jnp.int32, L)
    lane_f = lane.astype(jnp.float32)
    zeros16 = jnp.zeros((L,), jnp.float32)

    def group_core(wr, g, base_row):
        r_glob = base_row + wr
        r_f = r_glob.astype(jnp.float32)
        c0 = g * L
        fx = fxw[wr, pl.ds(c0, L)]
        fy = fyw[wr, pl.ds(c0, L)]
        jj = c0.astype(jnp.float32) + lane_f
        drf = _rne(fy + r_f)
        dcf = _rne(fx + jj)
        col_ok = (dcf >= 0.0) & (dcf < float(W))
        mine = (drf >= row0_f) & (drf < row0_f + float(ROWS_PER_TILE)) & col_ok
        off = (drf.astype(jnp.int32) - row0) * W + dcf.astype(jnp.int32)
        return r_f, c0, off, mine, col_ok, drf

    def scatter_vals(wr, c0, off, keep):
        plsc.store_scatter(vr, [off], imw[0, wr, pl.ds(c0, L)], mask=keep)
        plsc.store_scatter(vg, [off], imw[1, wr, pl.ds(c0, L)], mask=keep)
        plsc.store_scatter(vb, [off], imw[2, wr, pl.ds(c0, L)], mask=keep)

    def group_fast(wr, g, base_row, ofl, bad):
        r_f, c0, off, mine, col_ok, drf = group_core(wr, g, base_row)
        plsc.store_scatter(vr, [off], lane_f, mask=mine)
        w = plsc.load_gather(vr, [off], mask=mine)
        keep = mine & (w == lane_f)
        bad = bad | (mine & (w < lane_f)).astype(jnp.int32)
        scatter_vals(wr, c0, off, keep)
        r_glob = base_row + wr
        own = (r_glob >= row0) & (r_glob < row0 + ROWS_PER_TILE)
        far = ((drf >= 0.0) & (drf < float(H)) & col_ok
               & (jnp.abs(drf - r_f) > float(HALO)))
        ofl = ofl | jnp.where(own, far.astype(jnp.int32), 0)
        return ofl, bad

    def group_precise(wr, g, base_row):
        _, c0, off, mine, _, _ = group_core(wr, g, base_row)
        offu = jnp.where(mine, off, -1 - lane)
        dup_later = jnp.zeros((L,), jnp.bool_)
        for d in range(1, L):
            offd = _take16(offu, jnp.minimum(lane + d, L - 1))
            dup_later = dup_later | ((offd == offu) & (lane + d < L))
        keep = mine & jnp.logical_not(dup_later)
        scatter_vals(wr, c0, off, keep)

    def fast_rows(base_row, carry):
        def row_body(wr, c_in):
            def q_body(q, c2):
                ofl_in, bad_in = c2
                for u in range(UNROLL):
                    ofl_in, bad_in = group_fast(wr, q * UNROLL + u, base_row,
                                                ofl_in, bad_in)
                return ofl_in, bad_in
            return lax.fori_loop(0, GROUPS // UNROLL, q_body, c_in)
        return lax.fori_loop(0, WIN, row_body, carry)

    def precise_rows(base_row, nrows):
        def row_body(wr, _):
            def g_body(g, __):
                group_precise(wr, g, base_row)
                return 0
            return lax.fori_loop(0, GROUPS, g_body, 0)
        lax.fori_loop(0, nrows, row_body, 0)

    def zero_vals():
        def zbody(i, _):
            for u in range(UNROLL):
                base = (i * UNROLL + u) * L
                vr[pl.ds(base, L)] = zeros16
                vg[pl.ds(base, L)] = zeros16
                vb[pl.ds(base, L)] = zeros16
            return 0
        lax.fori_loop(0, TILE_ELEMS // (L * UNROLL), zbody, 0)

    def load_window_async(b, r_start, nrows):
        r_start = pl.multiple_of(r_start, 8)
        hs = []
        hs.append(pltpu.async_copy(x_hbm.at[b, 6, pl.ds(r_start, nrows)],
                                   fxw.at[pl.ds(0, nrows)], sem_win))
        hs.append(pltpu.async_copy(x_hbm.at[b, 7, pl.ds(r_start, nrows)],
                                   fyw.at[pl.ds(0, nrows)], sem_win))
        for ch in range(3):
            hs.append(pltpu.async_copy(x_hbm.at[b, ch, pl.ds(r_start, nrows)],
                                       imw.at[ch, pl.ds(0, nrows)], sem_win))
        return hs

    rband = pl.multiple_of(row0, 8)
    cp_hs = []
    for k in range(2):
        b = core * 2 + k
        for xch, och in ((0, 0), (1, 1), (2, 2), (3, 6), (4, 7), (5, 8),
                         (6, 9), (7, 10)):
            cp_hs.append(pltpu.async_copy(
                x_hbm.at[b, xch, pl.ds(rband, ROWS_PER_TILE)],
                cp_hbm.at[b, och, pl.ds(rband, ROWS_PER_TILE)], sem_cp))

    w0 = jnp.clip(row0 - HALO, 0, H - WIN)
    win_hs = load_window_async(core * 2, w0, WIN)

    for k in range(2):
        b = core * 2 + k
        zero_vals()
        for h in win_hs:
            h.wait()

        z16 = jnp.zeros((L,), jnp.int32)
        ofl, bad = fast_rows(w0, (z16, z16))

        flg_v[pl.ds(0, L)] = ofl
        pltpu.sync_copy(flg_v, shared.at[k, sub])
        plsc.subcore_barrier()
        pltpu.sync_copy(shared.at[k], flg_all)
        def or_body(i, a):
            return a | flg_all[i, pl.ds(0, L)]
        acc = lax.fori_loop(0, NSUB, or_body, jnp.zeros((L,), jnp.int32))
        any_out = jnp.any(acc != 0)
        any_bad = jnp.any(bad != 0)

        @pl.when(any_bad | any_out)
        def _fallback():
            zero_vals()
            def chunk_body(ci, _):
                r_start = pl.multiple_of(ci * FB_CHUNK, 8)
                pltpu.sync_copy(x_hbm.at[b, 6, pl.ds(r_start, FB_CHUNK)],
                                fxw.at[pl.ds(0, FB_CHUNK)])
                pltpu.sync_copy(x_hbm.at[b, 7, pl.ds(r_start, FB_CHUNK)],
                                fyw.at[pl.ds(0, FB_CHUNK)])
                for ch in range(3):
                    pltpu.sync_copy(x_hbm.at[b, ch, pl.ds(r_start, FB_CHUNK)],
                                    imw.at[ch, pl.ds(0, FB_CHUNK)])
                precise_rows(ci * FB_CHUNK, FB_CHUNK)
                return 0
            lax.fori_loop(0, H // FB_CHUNK, chunk_body, 0)

        if k == 0:
            win_hs = load_window_async(core * 2 + 1, w0, WIN)

        wh = []
        for ch, v in ((0, vr), (1, vg), (2, vb)):
            obase = pl.multiple_of((b * C + ch) * HW + row0 * W, 1024)
            wh.append(pltpu.async_copy(v, out_hbm.at[pl.ds(obase, TILE_ELEMS)],
                                       sem_w))
        for h in wh:
            h.wait()

    for h in cp_hs:
        h.wait()


def _sc_warp(x):
    mesh = plsc.VectorSubcoreMesh(core_axis_name="c", subcore_axis_name="s")
    fn = pl.kernel(
        _sc_warp_body,
        mesh=mesh,
        compiler_params=pltpu.CompilerParams(needs_layout_passes=False),
        out_type=[jax.ShapeDtypeStruct((B * C * HW,), jnp.float32),
                  jax.ShapeDtypeStruct((B, 12, H, W), jnp.float32)],
        scratch_types=[
            pltpu.VMEM((WIN, W), jnp.float32),
            pltpu.VMEM((WIN, W), jnp.float32),
            pltpu.VMEM((3, WIN, W), jnp.float32),
            pltpu.VMEM((TILE_ELEMS,), jnp.float32),
            pltpu.VMEM((TILE_ELEMS,), jnp.float32),
            pltpu.VMEM((TILE_ELEMS,), jnp.float32),
            pltpu.VMEM((L,), jnp.int32),
            pltpu.VMEM((NSUB, L), jnp.int32),
            pltpu.VMEM_SHARED((2, NSUB, L), jnp.int32),
            pltpu.SemaphoreType.DMA,
            pltpu.SemaphoreType.DMA,
            pltpu.SemaphoreType.DMA,
        ],
    )
    return fn(x)


def _tc_final_body(cp_ref, w_ref, o_ref):
    c = pl.program_id(2)

    @pl.when(c < 3)
    def _copy_warped():
        o_ref[0, 0] = w_ref[0, pl.ds(c, 1)][0]

    @pl.when(c == 3)
    def _err():
        d = w_ref[0] - cp_ref[0]
        o_ref[0, 0] = jnp.sqrt(d[0] * d[0] + d[1] * d[1] + d[2] * d[2])


def _tc_final(copies, warped):
    rows = 96
    grid = (B, H // rows, 4)
    return pl.pallas_call(
        _tc_final_body,
        grid=grid,
        in_specs=[
            pl.BlockSpec((1, 3, rows, W), lambda b, r, c: (b, 2, r, 0)),
            pl.BlockSpec((1, 3, rows, W), lambda b, r, c: (b, 0, r, 0)),
        ],
        out_specs=pl.BlockSpec(
            (1, 1, rows, W),
            lambda b, r, c: (b, jnp.where(c == 3, 11, c + 3), r, 0)),
        out_shape=jax.ShapeDtypeStruct((B, 12, H, W), jnp.float32),
        input_output_aliases={0: 0},
    )(copies, warped)


def kernel(x):
    warped_flat, copies = _sc_warp(x)
    return _tc_final(copies, warped_flat.reshape(B, C, H, W))


# speedup vs baseline: 4.6767x; 4.6767x over previous
import jax
import jax.numpy as jnp
from jax import lax
from jax.experimental import pallas as pl
from jax.experimental.pallas import tpu as pltpu
from jax.experimental.pallas import tpu_sc as plsc

B, C, H, W = 4, 3, 384, 384
HW = H * W
NCORES, NSUB, L = 2, 16, 16
ROWS_PER_TILE = H // NSUB
TILE_ELEMS = ROWS_PER_TILE * W
HALO = 8
WIN = ROWS_PER_TILE + 2 * HALO
FB_CHUNK = 32
GROUPS = W // L
UNROLL = 4
MAGIC = 12582912.0


def _rne(x):
    return (x + MAGIC) - MAGIC


def _take16(x, idx):
    return lax.gather(
        x, idx[:, None],
        lax.GatherDimensionNumbers(
            offset_dims=(), collapsed_slice_dims=(0,), start_index_map=(0,)),
        slice_sizes=(1,),
        mode=lax.GatherScatterMode.PROMISE_IN_BOUNDS)


def _sc_warp_body(x_hbm, out_hbm, fxw, fyw, imw, vr, vg, vb,
                  flg_v, flg_all, shared, sem_win, sem_w):
    core = lax.axis_index("c")
    sub = lax.axis_index("s")
    row0 = sub * ROWS_PER_TILE
    row0_f = row0.astype(jnp.float32)
    lane = lax.iota(jnp.int32, L)
    lane_f = lane.astype(jnp.float32)
    zeros16 = jnp.zeros((L,), jnp.float32)

    def group_core(wr, g, base_row):
        r_glob = base_row + wr
        r_f = r_glob.astype(jnp.float32)
        c0 = g * L
        fx = fxw[wr, pl.ds(c0, L)]
        fy = fyw[wr, pl.ds(c0, L)]
        jj = c0.astype(jnp.float32) + lane_f
        drf = _rne(fy + r_f)
        dcf = _rne(fx + jj)
        col_ok = (dcf >= 0.0) & (dcf < float(W))
        mine = (drf >= row0_f) & (drf < row0_f + float(ROWS_PER_TILE)) & col_ok
        off = (drf.astype(jnp.int32) - row0) * W + dcf.astype(jnp.int32)
        return r_f, c0, off, mine, col_ok, drf

    def scatter_vals(wr, c0, off, keep):
        plsc.store_scatter(vr, [off], imw[0, wr, pl.ds(c0, L)], mask=keep)
        plsc.store_scatter(vg, [off], imw[1, wr, pl.ds(c0, L)], mask=keep)
        plsc.store_scatter(vb, [off], imw[2, wr, pl.ds(c0, L)], mask=keep)

    def group_fast(wr, g, base_row, ofl, bad):
        r_f, c0, off, mine, col_ok, drf = group_core(wr, g, base_row)
        plsc.store_scatter(vr, [off], lane_f, mask=mine)
        w = plsc.load_gather(vr, [off], mask=mine)
        keep = mine & (w == lane_f)
        bad = bad | (mine & (w < lane_f)).astype(jnp.int32)
        scatter_vals(wr, c0, off, keep)
        r_glob = base_row + wr
        own = (r_glob >= row0) & (r_glob < row0 + ROWS_PER_TILE)
        far = ((drf >= 0.0) & (drf < float(H)) & col_ok
               & (jnp.abs(drf - r_f) > float(HALO)))
        ofl = ofl | jnp.where(own, far.astype(jnp.int32), 0)
        return ofl, bad

    def group_precise(wr, g, base_row):
        _, c0, off, mine, _, _ = group_core(wr, g, base_row)
        offu = jnp.where(mine, off, -1 - lane)
        dup_later = jnp.zeros((L,), jnp.bool_)
        for d in range(1, L):
            offd = _take16(offu, jnp.minimum(lane + d, L - 1))
            dup_later = dup_later | ((offd == offu) & (lane + d < L))
        keep = mine & jnp.logical_not(dup_later)
        scatter_vals(wr, c0, off, keep)

    def fast_rows(base_row, carry):
        def row_body(wr, c_in):
            def q_body(q, c2):
                ofl_in, bad_in = c2
                for u in range(UNROLL):
                    ofl_in, bad_in = group_fast(wr, q * UNROLL + u, base_row,
                                                ofl_in, bad_in)
                return ofl_in, bad_in
            return lax.fori_loop(0, GROUPS // UNROLL, q_body, c_in)
        return lax.fori_loop(0, WIN, row_body, carry)

    def precise_rows(base_row, nrows):
        def row_body(wr, _):
            def g_body(g, __):
                group_precise(wr, g, base_row)
                return 0
            return lax.fori_loop(0, GROUPS, g_body, 0)
        lax.fori_loop(0, nrows, row_body, 0)

    def zero_vals():
        def zbody(i, _):
            for u in range(UNROLL):
                base = (i * UNROLL + u) * L
                vr[pl.ds(base, L)] = zeros16
                vg[pl.ds(base, L)] = zeros16
                vb[pl.ds(base, L)] = zeros16
            return 0
        lax.fori_loop(0, TILE_ELEMS // (L * UNROLL), zbody, 0)

    def load_window_async(b, r_start, nrows):
        r_start = pl.multiple_of(r_start, 8)
        hs = []
        hs.append(pltpu.async_copy(x_hbm.at[b, 6, pl.ds(r_start, nrows)],
                                   fxw.at[pl.ds(0, nrows)], sem_win))
        hs.append(pltpu.async_copy(x_hbm.at[b, 7, pl.ds(r_start, nrows)],
                                   fyw.at[pl.ds(0, nrows)], sem_win))
        for ch in range(3):
            hs.append(pltpu.async_copy(x_hbm.at[b, ch, pl.ds(r_start, nrows)],
                                       imw.at[ch, pl.ds(0, nrows)], sem_win))
        return hs

    w0 = jnp.clip(row0 - HALO, 0, H - WIN)
    win_hs = load_window_async(core * 2, w0, WIN)

    for k in range(2):
        b = core * 2 + k
        zero_vals()
        for h in win_hs:
            h.wait()

        z16 = jnp.zeros((L,), jnp.int32)
        ofl, bad = fast_rows(w0, (z16, z16))

        flg_v[pl.ds(0, L)] = ofl
        pltpu.sync_copy(flg_v, shared.at[k, sub])
        plsc.subcore_barrier()
        pltpu.sync_copy(shared.at[k], flg_all)
        def or_body(i, a):
            return a | flg_all[i, pl.ds(0, L)]
        acc = lax.fori_loop(0, NSUB, or_body, jnp.zeros((L,), jnp.int32))
        any_out = jnp.any(acc != 0)
        any_bad = jnp.any(bad != 0)

        @pl.when(any_bad | any_out)
        def _fallback():
            zero_vals()
            def chunk_body(ci, _):
                r_start = pl.multiple_of(ci * FB_CHUNK, 8)
                pltpu.sync_copy(x_hbm.at[b, 6, pl.ds(r_start, FB_CHUNK)],
                                fxw.at[pl.ds(0, FB_CHUNK)])
                pltpu.sync_copy(x_hbm.at[b, 7, pl.ds(r_start, FB_CHUNK)],
                                fyw.at[pl.ds(0, FB_CHUNK)])
                for ch in range(3):
                    pltpu.sync_copy(x_hbm.at[b, ch, pl.ds(r_start, FB_CHUNK)],
                                    imw.at[ch, pl.ds(0, FB_CHUNK)])
                precise_rows(ci * FB_CHUNK, FB_CHUNK)
                return 0
            lax.fori_loop(0, H // FB_CHUNK, chunk_body, 0)

        if k == 0:
            win_hs = load_window_async(core * 2 + 1, w0, WIN)

        wh = []
        for ch, v in ((0, vr), (1, vg), (2, vb)):
            obase = pl.multiple_of((b * C + ch) * HW + row0 * W, 1024)
            wh.append(pltpu.async_copy(v, out_hbm.at[pl.ds(obase, TILE_ELEMS)],
                                       sem_w))
        for h in wh:
            h.wait()


def _sc_warp(x):
    mesh = plsc.VectorSubcoreMesh(core_axis_name="c", subcore_axis_name="s")
    fn = pl.kernel(
        _sc_warp_body,
        mesh=mesh,
        compiler_params=pltpu.CompilerParams(needs_layout_passes=False),
        out_type=jax.ShapeDtypeStruct((B * C * HW,), jnp.float32),
        scratch_types=[
            pltpu.VMEM((WIN, W), jnp.float32),
            pltpu.VMEM((WIN, W), jnp.float32),
            pltpu.VMEM((3, WIN, W), jnp.float32),
            pltpu.VMEM((TILE_ELEMS,), jnp.float32),
            pltpu.VMEM((TILE_ELEMS,), jnp.float32),
            pltpu.VMEM((TILE_ELEMS,), jnp.float32),
            pltpu.VMEM((L,), jnp.int32),
            pltpu.VMEM((NSUB, L), jnp.int32),
            pltpu.VMEM_SHARED((2, NSUB, L), jnp.int32),
            pltpu.SemaphoreType.DMA,
            pltpu.SemaphoreType.DMA,
        ],
    )
    return fn(x)


def _tc_err_body(x_ref, w_ref, o_ref):
    d = w_ref[0] - x_ref[0]
    o_ref[0, 0] = jnp.sqrt(d[0] * d[0] + d[1] * d[1] + d[2] * d[2])


def _tc_err(x, warped):
    rows = 96
    grid = (B, H // rows)
    return pl.pallas_call(
        _tc_err_body,
        grid=grid,
        in_specs=[
            pl.BlockSpec((1, 3, rows, W), lambda b, r: (b, 1, r, 0)),
            pl.BlockSpec((1, 3, rows, W), lambda b, r: (b, 0, r, 0)),
        ],
        out_specs=pl.BlockSpec((1, 1, rows, W), lambda b, r: (b, 0, r, 0)),
        out_shape=jax.ShapeDtypeStruct((B, 1, H, W), jnp.float32),
    )(x, warped)


def kernel(x):
    warped = _sc_warp(x).reshape(B, C, H, W)
    err = _tc_err(x, warped)
    return jnp.concatenate(
        [x[:, 0:3], warped, x[:, 3:6], x[:, 6:8], err], axis=1)


# speedup vs baseline: 5.2321x; 1.1187x over previous
import jax
import jax.numpy as jnp
from jax import lax
from jax.experimental import pallas as pl
from jax.experimental.pallas import tpu as pltpu
from jax.experimental.pallas import tpu_sc as plsc

B, C, H, W = 4, 3, 384, 384
HW = H * W
NCORES, NSUB, L = 2, 16, 16
ROWS_PER_TILE = H // NSUB
TILE_ELEMS = ROWS_PER_TILE * W
HALO = 8
WIN = ROWS_PER_TILE + 2 * HALO
FB_CHUNK = 32
GROUPS = W // L
UNROLL = 6
MAGIC = 12582912.0


def _rne(x):
    return (x + MAGIC) - MAGIC


def _take16(x, idx):
    return lax.gather(
        x, idx[:, None],
        lax.GatherDimensionNumbers(
            offset_dims=(), collapsed_slice_dims=(0,), start_index_map=(0,)),
        slice_sizes=(1,),
        mode=lax.GatherScatterMode.PROMISE_IN_BOUNDS)


def _sc_warp_body(x_hbm, out_hbm, fxw, fyw, imw, vr, vg, vb,
                  flg_v, flg_all, shared, sem_win, sem_w):
    core = lax.axis_index("c")
    sub = lax.axis_index("s")
    row0 = sub * ROWS_PER_TILE
    row0_f = row0.astype(jnp.float32)
    lane = lax.iota(jnp.int32, L)
    lane_f = lane.astype(jnp.float32)
    zeros16 = jnp.zeros((L,), jnp.float32)

    def group_core(wr, g, base_row):
        r_glob = base_row + wr
        r_f = r_glob.astype(jnp.float32)
        c0 = g * L
        fx = fxw[wr, pl.ds(c0, L)]
        fy = fyw[wr, pl.ds(c0, L)]
        jj = c0.astype(jnp.float32) + lane_f
        drf = _rne(fy + r_f)
        dcf = _rne(fx + jj)
        col_ok = (dcf >= 0.0) & (dcf < float(W))
        mine = (drf >= row0_f) & (drf < row0_f + float(ROWS_PER_TILE)) & col_ok
        off = (drf.astype(jnp.int32) - row0) * W + dcf.astype(jnp.int32)
        return r_f, c0, off, mine, col_ok, drf

    def scatter_vals(wr, c0, off, keep):
        plsc.store_scatter(vr, [off], imw[0, wr, pl.ds(c0, L)], mask=keep)
        plsc.store_scatter(vg, [off], imw[1, wr, pl.ds(c0, L)], mask=keep)
        plsc.store_scatter(vb, [off], imw[2, wr, pl.ds(c0, L)], mask=keep)

    def group_fast(wr, g, base_row, ofl, bad):
        r_f, c0, off, mine, col_ok, drf = group_core(wr, g, base_row)
        plsc.store_scatter(vr, [off], lane_f, mask=mine)
        w = plsc.load_gather(vr, [off], mask=mine)
        keep = mine & (w == lane_f)
        bad = bad | (mine & (w < lane_f)).astype(jnp.int32)
        scatter_vals(wr, c0, off, keep)
        r_glob = base_row + wr
        own = (r_glob >= row0) & (r_glob < row0 + ROWS_PER_TILE)
        far = jnp.abs(drf - r_f) > float(HALO)
        ofl = ofl | jnp.where(own, far.astype(jnp.int32), 0)
        return ofl, bad

    def group_precise(wr, g, base_row):
        _, c0, off, mine, _, _ = group_core(wr, g, base_row)
        offu = jnp.where(mine, off, -1 - lane)
        dup_later = jnp.zeros((L,), jnp.bool_)
        for d in range(1, L):
            offd = _take16(offu, jnp.minimum(lane + d, L - 1))
            dup_later = dup_later | ((offd == offu) & (lane + d < L))
        keep = mine & jnp.logical_not(dup_later)
        scatter_vals(wr, c0, off, keep)

    def fast_rows(base_row, carry):
        def row_body(wr, c_in):
            def q_body(q, c2):
                ofl_in, bad_in = c2
                for u in range(UNROLL):
                    ofl_in, bad_in = group_fast(wr, q * UNROLL + u, base_row,
                                                ofl_in, bad_in)
                return ofl_in, bad_in
            return lax.fori_loop(0, GROUPS // UNROLL, q_body, c_in)
        return lax.fori_loop(0, WIN, row_body, carry)

    def precise_rows(base_row, nrows):
        def row_body(wr, _):
            def g_body(g, __):
                group_precise(wr, g, base_row)
                return 0
            return lax.fori_loop(0, GROUPS, g_body, 0)
        lax.fori_loop(0, nrows, row_body, 0)

    def zero_vals():
        def zbody(i, _):
            for u in range(UNROLL):
                base = (i * UNROLL + u) * L
                vr[pl.ds(base, L)] = zeros16
                vg[pl.ds(base, L)] = zeros16
                vb[pl.ds(base, L)] = zeros16
            return 0
        lax.fori_loop(0, TILE_ELEMS // (L * UNROLL), zbody, 0)

    def load_window_async(b, r_start, nrows):
        r_start = pl.multiple_of(r_start, 8)
        hs = []
        hs.append(pltpu.async_copy(x_hbm.at[b, 6, pl.ds(r_start, nrows)],
                                   fxw.at[pl.ds(0, nrows)], sem_win))
        hs.append(pltpu.async_copy(x_hbm.at[b, 7, pl.ds(r_start, nrows)],
                                   fyw.at[pl.ds(0, nrows)], sem_win))
        for ch in range(3):
            hs.append(pltpu.async_copy(x_hbm.at[b, ch, pl.ds(r_start, nrows)],
                                       imw.at[ch, pl.ds(0, nrows)], sem_win))
        return hs

    w0 = jnp.clip(row0 - HALO, 0, H - WIN)
    win_hs = load_window_async(core * 2, w0, WIN)

    for k in range(2):
        b = core * 2 + k
        zero_vals()
        for h in win_hs:
            h.wait()

        z16 = jnp.zeros((L,), jnp.int32)
        ofl, bad = fast_rows(w0, (z16, z16))

        flg_v[pl.ds(0, L)] = ofl
        pltpu.sync_copy(flg_v, shared.at[k, sub])
        plsc.subcore_barrier()
        pltpu.sync_copy(shared.at[k], flg_all)
        def or_body(i, a):
            return a | flg_all[i, pl.ds(0, L)]
        acc = lax.fori_loop(0, NSUB, or_body, jnp.zeros((L,), jnp.int32))
        any_out = jnp.any(acc != 0)
        any_bad = jnp.any(bad != 0)

        @pl.when(any_bad | any_out)
        def _fallback():
            zero_vals()
            def chunk_body(ci, _):
                r_start = pl.multiple_of(ci * FB_CHUNK, 8)
                pltpu.sync_copy(x_hbm.at[b, 6, pl.ds(r_start, FB_CHUNK)],
                                fxw.at[pl.ds(0, FB_CHUNK)])
                pltpu.sync_copy(x_hbm.at[b, 7, pl.ds(r_start, FB_CHUNK)],
                                fyw.at[pl.ds(0, FB_CHUNK)])
                for ch in range(3):
                    pltpu.sync_copy(x_hbm.at[b, ch, pl.ds(r_start, FB_CHUNK)],
                                    imw.at[ch, pl.ds(0, FB_CHUNK)])
                precise_rows(ci * FB_CHUNK, FB_CHUNK)
                return 0
            lax.fori_loop(0, H // FB_CHUNK, chunk_body, 0)

        if k == 0:
            win_hs = load_window_async(core * 2 + 1, w0, WIN)

        wh = []
        for ch, v in ((0, vr), (1, vg), (2, vb)):
            obase = pl.multiple_of((b * C + ch) * HW + row0 * W, 1024)
            wh.append(pltpu.async_copy(v, out_hbm.at[pl.ds(obase, TILE_ELEMS)],
                                       sem_w))
        for h in wh:
            h.wait()


def _sc_warp(x):
    mesh = plsc.VectorSubcoreMesh(core_axis_name="c", subcore_axis_name="s")
    fn = pl.kernel(
        _sc_warp_body,
        mesh=mesh,
        compiler_params=pltpu.CompilerParams(needs_layout_passes=False),
        out_type=jax.ShapeDtypeStruct((B * C * HW,), jnp.float32),
        scratch_types=[
            pltpu.VMEM((WIN, W), jnp.float32),
            pltpu.VMEM((WIN, W), jnp.float32),
            pltpu.VMEM((3, WIN, W), jnp.float32),
            pltpu.VMEM((TILE_ELEMS,), jnp.float32),
            pltpu.VMEM((TILE_ELEMS,), jnp.float32),
            pltpu.VMEM((TILE_ELEMS,), jnp.float32),
            pltpu.VMEM((L,), jnp.int32),
            pltpu.VMEM((NSUB, L), jnp.int32),
            pltpu.VMEM_SHARED((2, NSUB, L), jnp.int32),
            pltpu.SemaphoreType.DMA,
            pltpu.SemaphoreType.DMA,
        ],
    )
    return fn(x)


def _tc_assemble_body(x_ref, w_ref, o_ref):
    a = x_ref[0]
    w = w_ref[0]
    im2 = a[3:6]
    d = w - im2
    err = jnp.sqrt(d[0] * d[0] + d[1] * d[1] + d[2] * d[2])
    o_ref[0, 0:3] = a[0:3]
    o_ref[0, 3:6] = w
    o_ref[0, 6:9] = im2
    o_ref[0, 9:11] = a[6:8]
    o_ref[0, 11] = err


def _tc_assemble(x, warped):
    rows = 96
    grid = (B, H // rows)
    return pl.pallas_call(
        _tc_assemble_body,
        grid=grid,
        in_specs=[
            pl.BlockSpec((1, 8, rows, W), lambda b, r: (b, 0, r, 0)),
            pl.BlockSpec((1, 3, rows, W), lambda b, r: (b, 0, r, 0)),
        ],
        out_specs=pl.BlockSpec((1, 12, rows, W), lambda b, r: (b, 0, r, 0)),
        out_shape=jax.ShapeDtypeStruct((B, 12, H, W), jnp.float32),
    )(x, warped)


def kernel(x):
    warped = _sc_warp(x).reshape(B, C, H, W)
    return _tc_assemble(x, warped)


# speedup vs baseline: 5.3515x; 1.0228x over previous
import jax
import jax.numpy as jnp
from jax import lax
from jax.experimental import pallas as pl
from jax.experimental.pallas import tpu as pltpu
from jax.experimental.pallas import tpu_sc as plsc

B, C, H, W = 4, 3, 384, 384
HW = H * W
NCORES, NSUB, L = 2, 16, 16
ROWS_PER_TILE = H // NSUB
TILE_ELEMS = ROWS_PER_TILE * W
HALO = 8
WIN = ROWS_PER_TILE + 2 * HALO
FB_CHUNK = 32
GROUPS = W // L
UNROLL = 6
MAGIC = 12582912.0


def _rne(x):
    return (x + MAGIC) - MAGIC


def _take16(x, idx):
    return lax.gather(
        x, idx[:, None],
        lax.GatherDimensionNumbers(
            offset_dims=(), collapsed_slice_dims=(0,), start_index_map=(0,)),
        slice_sizes=(1,),
        mode=lax.GatherScatterMode.PROMISE_IN_BOUNDS)


def _sc_warp_body(x_hbm, out_hbm, fxw, fyw, imw, vr, vg, vb, wk,
                  flg_v, flg_all, shared, sem_win, sem_w):
    core = lax.axis_index("c")
    sub = lax.axis_index("s")
    row0 = sub * ROWS_PER_TILE
    row0_f = row0.astype(jnp.float32)
    lane = lax.iota(jnp.int32, L)
    lane_f = lane.astype(jnp.float32)
    zeros16 = jnp.zeros((L,), jnp.float32)

    def group_core(wr, g, base_row):
        r_glob = base_row + wr
        r_f = r_glob.astype(jnp.float32)
        c0 = g * L
        fx = fxw[wr, pl.ds(c0, L)]
        fy = fyw[wr, pl.ds(c0, L)]
        jj = c0.astype(jnp.float32) + lane_f
        drf = _rne(fy + r_f)
        dcf = _rne(fx + jj)
        col_ok = (dcf >= 0.0) & (dcf < float(W))
        mine = (drf >= row0_f) & (drf < row0_f + float(ROWS_PER_TILE)) & col_ok
        off = (drf.astype(jnp.int32) - row0) * W + dcf.astype(jnp.int32)
        return r_f, c0, off, mine, col_ok, drf

    def scatter_vals(wr, c0, off, keep):
        plsc.store_scatter(vr, [off], imw[0, wr, pl.ds(c0, L)], mask=keep)
        plsc.store_scatter(vg, [off], imw[1, wr, pl.ds(c0, L)], mask=keep)
        plsc.store_scatter(vb, [off], imw[2, wr, pl.ds(c0, L)], mask=keep)

    def group_fast(wr, g, base_row, ofl, bad):
        r_f, c0, off, mine, col_ok, drf = group_core(wr, g, base_row)
        plsc.store_scatter(wk, [off], lane_f, mask=mine)
        w = plsc.load_gather(wk, [off], mask=mine)
        keep = mine & (w == lane_f)
        bad = bad | (mine & (w < lane_f)).astype(jnp.int32)
        scatter_vals(wr, c0, off, keep)
        r_glob = base_row + wr
        own = (r_glob >= row0) & (r_glob < row0 + ROWS_PER_TILE)
        far = jnp.abs(drf - r_f) > float(HALO)
        ofl = ofl | jnp.where(own, far.astype(jnp.int32), 0)
        return ofl, bad

    def group_precise(wr, g, base_row):
        _, c0, off, mine, _, _ = group_core(wr, g, base_row)
        offu = jnp.where(mine, off, -1 - lane)
        dup_later = jnp.zeros((L,), jnp.bool_)
        for d in range(1, L):
            offd = _take16(offu, jnp.minimum(lane + d, L - 1))
            dup_later = dup_later | ((offd == offu) & (lane + d < L))
        keep = mine & jnp.logical_not(dup_later)
        scatter_vals(wr, c0, off, keep)

    def fast_rows(base_row, carry):
        def row_body(wr, c_in):
            def q_body(q, c2):
                ofl_in, bad_in = c2
                for u in range(UNROLL):
                    ofl_in, bad_in = group_fast(wr, q * UNROLL + u, base_row,
                                                ofl_in, bad_in)
                return ofl_in, bad_in
            return lax.fori_loop(0, GROUPS // UNROLL, q_body, c_in)
        return lax.fori_loop(0, WIN, row_body, carry)

    def precise_rows(base_row, nrows):
        def row_body(wr, _):
            def g_body(g, __):
                group_precise(wr, g, base_row)
                return 0
            return lax.fori_loop(0, GROUPS, g_body, 0)
        lax.fori_loop(0, nrows, row_body, 0)

    def zero_vals():
        def zbody(i, _):
            for u in range(UNROLL):
                base = (i * UNROLL + u) * L
                vr[pl.ds(base, L)] = zeros16
                vg[pl.ds(base, L)] = zeros16
                vb[pl.ds(base, L)] = zeros16
            return 0
        lax.fori_loop(0, TILE_ELEMS // (L * UNROLL), zbody, 0)

    def load_window_async(b, r_start, nrows):
        r_start = pl.multiple_of(r_start, 8)
        hs = []
        hs.append(pltpu.async_copy(x_hbm.at[b, 6, pl.ds(r_start, nrows)],
                                   fxw.at[pl.ds(0, nrows)], sem_win))
        hs.append(pltpu.async_copy(x_hbm.at[b, 7, pl.ds(r_start, nrows)],
                                   fyw.at[pl.ds(0, nrows)], sem_win))
        for ch in range(3):
            hs.append(pltpu.async_copy(x_hbm.at[b, ch, pl.ds(r_start, nrows)],
                                       imw.at[ch, pl.ds(0, nrows)], sem_win))
        return hs

    w0 = jnp.clip(row0 - HALO, 0, H - WIN)
    win_hs = load_window_async(core * 2, w0, WIN)

    for k in range(2):
        b = core * 2 + k
        zero_vals()
        for h in win_hs:
            h.wait()

        z16 = jnp.zeros((L,), jnp.int32)
        ofl, bad = fast_rows(w0, (z16, z16))

        flg_v[pl.ds(0, L)] = ofl
        pltpu.sync_copy(flg_v, shared.at[k, sub])
        plsc.subcore_barrier()
        pltpu.sync_copy(shared.at[k], flg_all)
        def or_body(i, a):
            return a | flg_all[i, pl.ds(0, L)]
        acc = lax.fori_loop(0, NSUB, or_body, jnp.zeros((L,), jnp.int32))
        any_out = jnp.any(acc != 0)
        any_bad = jnp.any(bad != 0)

        @pl.when(any_bad | any_out)
        def _fallback():
            zero_vals()
            def chunk_body(ci, _):
                r_start = pl.multiple_of(ci * FB_CHUNK, 8)
                pltpu.sync_copy(x_hbm.at[b, 6, pl.ds(r_start, FB_CHUNK)],
                                fxw.at[pl.ds(0, FB_CHUNK)])
                pltpu.sync_copy(x_hbm.at[b, 7, pl.ds(r_start, FB_CHUNK)],
                                fyw.at[pl.ds(0, FB_CHUNK)])
                for ch in range(3):
                    pltpu.sync_copy(x_hbm.at[b, ch, pl.ds(r_start, FB_CHUNK)],
                                    imw.at[ch, pl.ds(0, FB_CHUNK)])
                precise_rows(ci * FB_CHUNK, FB_CHUNK)
                return 0
            lax.fori_loop(0, H // FB_CHUNK, chunk_body, 0)

        if k == 0:
            win_hs = load_window_async(core * 2 + 1, w0, WIN)

        wh = []
        for ch, v in ((0, vr), (1, vg), (2, vb)):
            obase = pl.multiple_of((b * C + ch) * HW + row0 * W, 1024)
            wh.append(pltpu.async_copy(v, out_hbm.at[pl.ds(obase, TILE_ELEMS)],
                                       sem_w))
        for h in wh:
            h.wait()


def _sc_warp(x):
    mesh = plsc.VectorSubcoreMesh(core_axis_name="c", subcore_axis_name="s")
    fn = pl.kernel(
        _sc_warp_body,
        mesh=mesh,
        compiler_params=pltpu.CompilerParams(needs_layout_passes=False),
        out_type=jax.ShapeDtypeStruct((B * C * HW,), jnp.float32),
        scratch_types=[
            pltpu.VMEM((WIN, W), jnp.float32),
            pltpu.VMEM((WIN, W), jnp.float32),
            pltpu.VMEM((3, WIN, W), jnp.float32),
            pltpu.VMEM((TILE_ELEMS,), jnp.float32),
            pltpu.VMEM((TILE_ELEMS,), jnp.float32),
            pltpu.VMEM((TILE_ELEMS,), jnp.float32),
            pltpu.VMEM((TILE_ELEMS,), jnp.float32),
            pltpu.VMEM((L,), jnp.int32),
            pltpu.VMEM((NSUB, L), jnp.int32),
            pltpu.VMEM_SHARED((2, NSUB, L), jnp.int32),
            pltpu.SemaphoreType.DMA,
            pltpu.SemaphoreType.DMA,
        ],
    )
    return fn(x)


def _tc_assemble_body(x_ref, w_ref, o_ref):
    a = x_ref[0]
    w = w_ref[0]
    im2 = a[3:6]
    d = w - im2
    err = jnp.sqrt(d[0] * d[0] + d[1] * d[1] + d[2] * d[2])
    o_ref[0, 0:3] = a[0:3]
    o_ref[0, 3:6] = w
    o_ref[0, 6:9] = im2
    o_ref[0, 9:11] = a[6:8]
    o_ref[0, 11] = err


def _tc_assemble(x, warped):
    rows = 192
    grid = (B, H // rows)
    return pl.pallas_call(
        _tc_assemble_body,
        grid=grid,
        in_specs=[
            pl.BlockSpec((1, 8, rows, W), lambda b, r: (b, 0, r, 0)),
            pl.BlockSpec((1, 3, rows, W), lambda b, r: (b, 0, r, 0)),
        ],
        out_specs=pl.BlockSpec((1, 12, rows, W), lambda b, r: (b, 0, r, 0)),
        out_shape=jax.ShapeDtypeStruct((B, 12, H, W), jnp.float32),
    )(x, warped)


def kernel(x):
    warped = _sc_warp(x).reshape(B, C, H, W)
    return _tc_assemble(x, warped)
